# Initial kernel scaffold; baseline (speedup 1.0000x reference)
import jax
import jax.numpy as jnp
from jax.experimental import pallas as pl


def kernel(kx, vx, k_cache, v_cache):
    raise NotImplementedError("write your pallas kernel here")


# speedup vs baseline: 67.8066x; 67.8066x over previous
import jax
import jax.numpy as jnp
from jax.experimental import pallas as pl


def _copy_kernel(kx_ref, vx_ref, k_out_ref, v_out_ref):
    k_out_ref[...] = kx_ref[...]
    v_out_ref[...] = vx_ref[...]


def kernel(kx, vx, k_cache, v_cache):
    del k_cache, v_cache
    out_shape = (
        jax.ShapeDtypeStruct(kx.shape, kx.dtype),
        jax.ShapeDtypeStruct(vx.shape, vx.dtype),
    )
    return pl.pallas_call(
        _copy_kernel,
        out_shape=out_shape,
    )(kx, vx)
